# baseline (speedup 1.0000x reference)
import functools

import jax
import jax.numpy as jnp
from jax import lax
from jax.experimental import pallas as pl
from jax.experimental.pallas import tpu as pltpu
from jax.experimental.pallas import tpu_sc as plsc

B = 16384
D = 128
NC = 2
NS = 16
NW = NC * NS
B_PER_W = B // NW
CHUNK = 128
N_CHUNKS = B_PER_W // CHUNK
ROWS = 288
REP = 8
RPC = REP // NC
L = 16


def _gather_body(idx_hbm, table_hbm, out_hbm, rep_hbm, idx_v, tbl_v, rows_v, sem):
    c = lax.axis_index("c")
    s = lax.axis_index("s")
    wid = s * NC + c
    base = wid * B_PER_W

    pltpu.sync_copy(idx_hbm.at[pl.ds(wid * N_CHUNKS, N_CHUNKS)], idx_v)

    @pl.when(s < RPC)
    def _build():
        pltpu.sync_copy(table_hbm, tbl_v)
        pltpu.sync_copy(tbl_v, rep_hbm.at[pl.ds((c * RPC + s) * ROWS, ROWS)])

    rep_off = (c * RPC + s % RPC) * ROWS
    offv = jnp.full((L,), rep_off, dtype=jnp.int32)
    for j in range(N_CHUNKS):
        for k in range(CHUNK // L):
            idx_v[j, pl.ds(k * L, L)] = idx_v[j, pl.ds(k * L, L)] + offv

    plsc.subcore_barrier()

    descs = [
        pltpu.async_copy(
            rep_hbm.at[idx_v.at[j]],
            rows_v.at[pl.ds(j * CHUNK, CHUNK)],
            sem,
        )
        for j in range(N_CHUNKS)
    ]
    for d in descs:
        d.wait()
    pltpu.sync_copy(rows_v, out_hbm.at[pl.ds(base, B_PER_W)])


@jax.jit
def kernel(batch_rels, table):
    idx = batch_rels.astype(jnp.int32).reshape(NW * N_CHUNKS, CHUNK)
    mesh = plsc.VectorSubcoreMesh(
        core_axis_name="c", subcore_axis_name="s", num_cores=NC, num_subcores=NS
    )
    f = pl.kernel(
        _gather_body,
        out_type=(
            jax.ShapeDtypeStruct((B, D), jnp.float32),
            jax.ShapeDtypeStruct((REP * ROWS, D), jnp.float32),
        ),
        mesh=mesh,
        scratch_types=[
            pltpu.VMEM((N_CHUNKS, CHUNK), jnp.int32),
            pltpu.VMEM((ROWS, D), jnp.float32),
            pltpu.VMEM((B_PER_W, D), jnp.float32),
            pltpu.SemaphoreType.DMA,
        ],
    )
    out, _ = f(idx, table)
    return out


# speedup vs baseline: 1.0995x; 1.0995x over previous
import functools

import jax
import jax.numpy as jnp
from jax import lax
from jax.experimental import pallas as pl
from jax.experimental.pallas import tpu as pltpu
from jax.experimental.pallas import tpu_sc as plsc

B = 16384
D = 128
NC = 2
NS = 16
NW = NC * NS
B_PER_W = B // NW
CHUNK = 128
N_CHUNKS = B_PER_W // CHUNK
ROWS = 288
REP = 8
RPC = REP // NC
L = 16
SUB = 24


def _gather_body(idx_hbm, table_hbm, out_hbm, rep_hbm, idx_v, tbl_v, rows_v, sem, bsem):
    c = lax.axis_index("c")
    s = lax.axis_index("s")
    wid = s * NC + c
    base = wid * B_PER_W
    d_idx = pltpu.async_copy(
        idx_hbm.at[pl.ds(wid * N_CHUNKS, N_CHUNKS)], idx_v, sem
    )

    @pl.when(s < ROWS // SUB)
    def _build():
        pltpu.sync_copy(table_hbm.at[pl.ds(s * SUB, SUB)], tbl_v)
        writes = [
            pltpu.async_copy(
                tbl_v,
                rep_hbm.at[pl.ds((c * RPC + r) * ROWS + s * SUB, SUB)],
                bsem,
            )
            for r in range(RPC)
        ]
        for w in writes:
            w.wait()

    d_idx.wait()

    rep_off = (c * RPC + s % RPC) * ROWS
    offv = jnp.full((L,), rep_off, dtype=jnp.int32)
    for j in range(N_CHUNKS):
        for k in range(CHUNK // L):
            idx_v[j, pl.ds(k * L, L)] = idx_v[j, pl.ds(k * L, L)] + offv

    plsc.subcore_barrier()

    descs = [
        pltpu.async_copy(
            rep_hbm.at[idx_v.at[j]],
            rows_v.at[pl.ds(j * CHUNK, CHUNK)],
            sem,
        )
        for j in range(N_CHUNKS)
    ]
    for d in descs:
        d.wait()
    pltpu.sync_copy(rows_v, out_hbm.at[pl.ds(base, B_PER_W)])


@jax.jit
def kernel(batch_rels, table):
    idx = batch_rels.astype(jnp.int32).reshape(NW * N_CHUNKS, CHUNK)
    mesh = plsc.VectorSubcoreMesh(
        core_axis_name="c", subcore_axis_name="s", num_cores=NC, num_subcores=NS
    )
    f = pl.kernel(
        _gather_body,
        out_type=(
            jax.ShapeDtypeStruct((B, D), jnp.float32),
            jax.ShapeDtypeStruct((REP * ROWS, D), jnp.float32),
        ),
        mesh=mesh,
        scratch_types=[
            pltpu.VMEM((N_CHUNKS, CHUNK), jnp.int32),
            pltpu.VMEM((SUB, D), jnp.float32),
            pltpu.VMEM((B_PER_W, D), jnp.float32),
            pltpu.SemaphoreType.DMA,
            pltpu.SemaphoreType.DMA,
        ],
    )
    out, _ = f(idx, table)
    return out


# speedup vs baseline: 1.1270x; 1.0250x over previous
import functools

import jax
import jax.numpy as jnp
from jax import lax
from jax.experimental import pallas as pl
from jax.experimental.pallas import tpu as pltpu
from jax.experimental.pallas import tpu_sc as plsc

B = 16384
D = 128
NC = 2
NS = 16
NW = NC * NS
B_PER_W = B // NW
CHUNK = 128
N_CHUNKS = B_PER_W // CHUNK
ROWS = 288
REP = 16
RPC = REP // NC
L = 16
SUB = 24


def _gather_body(idx_hbm, table_hbm, out_hbm, rep_hbm, idx_v, tbl_v, rows_v, sem, bsem):
    c = lax.axis_index("c")
    s = lax.axis_index("s")
    wid = s * NC + c
    base = wid * B_PER_W
    d_idx = pltpu.async_copy(
        idx_hbm.at[pl.ds(wid * N_CHUNKS, N_CHUNKS)], idx_v, sem
    )

    @pl.when(s < ROWS // SUB)
    def _build():
        pltpu.sync_copy(table_hbm.at[pl.ds(s * SUB, SUB)], tbl_v)
        writes = [
            pltpu.async_copy(
                tbl_v,
                rep_hbm.at[pl.ds((c * RPC + r) * ROWS + s * SUB, SUB)],
                bsem,
            )
            for r in range(RPC)
        ]
        for w in writes:
            w.wait()

    d_idx.wait()

    rep_off = (c * RPC + s % RPC) * ROWS
    offv = jnp.full((L,), rep_off, dtype=jnp.int32)
    for j in range(N_CHUNKS):
        for k in range(CHUNK // L):
            idx_v[j, pl.ds(k * L, L)] = idx_v[j, pl.ds(k * L, L)] + offv

    plsc.subcore_barrier()

    descs = [
        pltpu.async_copy(
            rep_hbm.at[idx_v.at[j]],
            rows_v.at[pl.ds(j * CHUNK, CHUNK)],
            sem,
        )
        for j in range(N_CHUNKS)
    ]
    for d in descs:
        d.wait()
    pltpu.sync_copy(rows_v, out_hbm.at[pl.ds(base, B_PER_W)])


@jax.jit
def kernel(batch_rels, table):
    idx = batch_rels.astype(jnp.int32).reshape(NW * N_CHUNKS, CHUNK)
    mesh = plsc.VectorSubcoreMesh(
        core_axis_name="c", subcore_axis_name="s", num_cores=NC, num_subcores=NS
    )
    f = pl.kernel(
        _gather_body,
        out_type=(
            jax.ShapeDtypeStruct((B, D), jnp.float32),
            jax.ShapeDtypeStruct((REP * ROWS, D), jnp.float32),
        ),
        mesh=mesh,
        scratch_types=[
            pltpu.VMEM((N_CHUNKS, CHUNK), jnp.int32),
            pltpu.VMEM((SUB, D), jnp.float32),
            pltpu.VMEM((B_PER_W, D), jnp.float32),
            pltpu.SemaphoreType.DMA,
            pltpu.SemaphoreType.DMA,
        ],
    )
    out, _ = f(idx, table)
    return out
